# baseline (speedup 1.0000x reference)
import jax
import jax.numpy as jnp
from jax.experimental import pallas as pl

B, S, D = 1, 2048, 2048
E, K = 8, 2
M = 4096
G = 4
GS = (B * S) // G
CAP = int(GS * K / E * 1.25)

_HI = jax.lax.Precision.HIGHEST


def _route(xg, wr):
    logits = jax.lax.dot_general(xg, wr, (((1,), (0,)), ((), ())),
                                 preferred_element_type=jnp.float32)
    lmax = jnp.max(logits, axis=1, keepdims=True)
    ex = jnp.exp(logits - lmax)
    probs = ex / jnp.sum(ex, axis=1, keepdims=True)

    iota_e = jax.lax.broadcasted_iota(jnp.int32, (GS, E), 1)
    m1 = jnp.max(probs, axis=1, keepdims=True)
    a1 = jnp.min(jnp.where(probs == m1, iota_e, E), axis=1, keepdims=True)
    probs2 = jnp.where(iota_e == a1, -jnp.inf, probs)
    m2 = jnp.max(probs2, axis=1, keepdims=True)
    a2 = jnp.min(jnp.where(probs2 == m2, iota_e, E), axis=1, keepdims=True)

    oh0 = (iota_e == a1).astype(jnp.float32)
    oh1 = (iota_e == a2).astype(jnp.float32)
    ri = jax.lax.broadcasted_iota(jnp.int32, (GS, GS), 0)
    ci = jax.lax.broadcasted_iota(jnp.int32, (GS, GS), 1)
    tri = (ci <= ri).astype(jnp.float32)
    c0 = jax.lax.dot_general(tri, oh0, (((1,), (0,)), ((), ())),
                             preferred_element_type=jnp.float32)
    c1 = jax.lax.dot_general(tri, oh1, (((1,), (0,)), ((), ())),
                             preferred_element_type=jnp.float32)
    total0 = c0[GS - 1:GS, :]
    pos0 = jnp.sum(c0 * oh0, axis=1, keepdims=True) - 1.0
    pos1 = jnp.sum((c1 + total0) * oh1, axis=1, keepdims=True) - 1.0
    return a1, a2, m1, m2, pos0, pos1


def _router_body(xg_ref, wr_ref, xe_ref, comb_ref):
    e = pl.program_id(1)
    xg = xg_ref[0]
    a1, a2, m1, m2, pos0, pos1 = _route(xg, wr_ref[...])

    iota_c = jax.lax.broadcasted_iota(jnp.int32, (GS, CAP), 1)
    hit0 = (a1 == e) & (iota_c == pos0.astype(jnp.int32)) & (pos0 < CAP)
    hit1 = (a2 == e) & (iota_c == pos1.astype(jnp.int32)) & (pos1 < CAP)
    disp = hit0.astype(jnp.float32) + hit1.astype(jnp.float32)
    comb_ref[0, 0] = jnp.where(hit0, m1, 0.0) + jnp.where(hit1, m2, 0.0)
    xe_ref[0] = jax.lax.dot_general(disp, xg, (((0,), (0,)), ((), ())),
                                    preferred_element_type=jnp.float32,
                                    precision=_HI)


def _ffn_body(xe_ref, w0_ref, w1_ref, wo_ref, oe_ref):
    mt = pl.program_id(1)
    a = xe_ref[0]
    h0 = jax.lax.dot_general(a, w0_ref[0], (((1,), (0,)), ((), ())),
                             preferred_element_type=jnp.float32, precision=_HI)
    h1 = jax.lax.dot_general(a, w1_ref[0], (((1,), (0,)), ((), ())),
                             preferred_element_type=jnp.float32, precision=_HI)
    h = (h0 * jax.lax.logistic(h0)) * h1
    out = jax.lax.dot_general(h, wo_ref[0], (((1,), (0,)), ((), ())),
                              preferred_element_type=jnp.float32, precision=_HI)

    @pl.when(mt == 0)
    def _():
        oe_ref[0] = out

    @pl.when(mt > 0)
    def _():
        oe_ref[0] += out


def _combine_body(comb_ref, oe_ref, y_ref):
    e = pl.program_id(1)
    y = jax.lax.dot_general(comb_ref[0, 0], oe_ref[0],
                            (((1,), (0,)), ((), ())),
                            preferred_element_type=jnp.float32,
                            precision=_HI)

    @pl.when(e == 0)
    def _():
        y_ref[0] = y

    @pl.when(e > 0)
    def _():
        y_ref[0] += y


BM = 256


@jax.jit
def kernel(x, w_router, wi_0, wi_1, wo):
    xg = x.reshape(G, GS, D)

    xe, comb = pl.pallas_call(
        _router_body,
        grid=(G, E),
        in_specs=[
            pl.BlockSpec((1, GS, D), lambda g, e: (g, 0, 0)),
            pl.BlockSpec((D, E), lambda g, e: (0, 0)),
        ],
        out_specs=[
            pl.BlockSpec((1, CAP, D), lambda g, e: (e, g, 0)),
            pl.BlockSpec((1, 1, GS, CAP), lambda g, e: (g, e, 0, 0)),
        ],
        out_shape=[
            jax.ShapeDtypeStruct((E, G * CAP, D), jnp.float32),
            jax.ShapeDtypeStruct((G, E, GS, CAP), jnp.float32),
        ],
    )(xg, w_router)

    oe = pl.pallas_call(
        _ffn_body,
        grid=(E, M // BM),
        in_specs=[
            pl.BlockSpec((1, G * CAP, D), lambda e, mt: (e, 0, 0)),
            pl.BlockSpec((1, D, BM), lambda e, mt: (e, 0, mt)),
            pl.BlockSpec((1, D, BM), lambda e, mt: (e, 0, mt)),
            pl.BlockSpec((1, BM, D), lambda e, mt: (e, mt, 0)),
        ],
        out_specs=pl.BlockSpec((1, G * CAP, D), lambda e, mt: (e, 0, 0)),
        out_shape=jax.ShapeDtypeStruct((E, G * CAP, D), jnp.float32),
    )(xe, wi_0, wi_1, wo)

    y = pl.pallas_call(
        _combine_body,
        grid=(G, E),
        in_specs=[
            pl.BlockSpec((1, 1, GS, CAP), lambda g, e: (g, e, 0, 0)),
            pl.BlockSpec((1, CAP, D), lambda g, e: (e, g, 0)),
        ],
        out_specs=pl.BlockSpec((1, GS, D), lambda g, e: (g, 0, 0)),
        out_shape=jax.ShapeDtypeStruct((G, GS, D), jnp.float32),
    )(comb, oe)

    return y.reshape(B, S, D)


# speedup vs baseline: 2.8968x; 2.8968x over previous
import jax
import jax.numpy as jnp
from jax.experimental import pallas as pl

B, S, D = 1, 2048, 2048
E, K = 8, 2
M = 4096
G = 4
GS = (B * S) // G
CAP = int(GS * K / E * 1.25)

_HI = jax.lax.Precision.HIGHEST


def _route(xg, wr):
    logits = jax.lax.dot_general(xg, wr, (((1,), (0,)), ((), ())),
                                 preferred_element_type=jnp.float32)
    lmax = jnp.max(logits, axis=1, keepdims=True)
    ex = jnp.exp(logits - lmax)
    probs = ex / jnp.sum(ex, axis=1, keepdims=True)

    iota_e = jax.lax.broadcasted_iota(jnp.int32, (GS, E), 1)
    m1 = jnp.max(probs, axis=1, keepdims=True)
    a1 = jnp.min(jnp.where(probs == m1, iota_e, E), axis=1, keepdims=True)
    probs2 = jnp.where(iota_e == a1, -jnp.inf, probs)
    m2 = jnp.max(probs2, axis=1, keepdims=True)
    a2 = jnp.min(jnp.where(probs2 == m2, iota_e, E), axis=1, keepdims=True)

    oh0 = (iota_e == a1).astype(jnp.float32)
    oh1 = (iota_e == a2).astype(jnp.float32)
    ri = jax.lax.broadcasted_iota(jnp.int32, (GS, GS), 0)
    ci = jax.lax.broadcasted_iota(jnp.int32, (GS, GS), 1)
    tri = (ci <= ri).astype(jnp.float32)
    c0 = jax.lax.dot_general(tri, oh0, (((1,), (0,)), ((), ())),
                             preferred_element_type=jnp.float32)
    c1 = jax.lax.dot_general(tri, oh1, (((1,), (0,)), ((), ())),
                             preferred_element_type=jnp.float32)
    total0 = c0[GS - 1:GS, :]
    pos0 = jnp.sum(c0 * oh0, axis=1, keepdims=True) - 1.0
    pos1 = jnp.sum((c1 + total0) * oh1, axis=1, keepdims=True) - 1.0
    return a1, a2, m1, m2, pos0, pos1


def _router_body(xg_ref, wr_ref, xe_ref, comb_ref):
    e = pl.program_id(1)
    xg = xg_ref[0]
    a1, a2, m1, m2, pos0, pos1 = _route(xg, wr_ref[...])

    iota_c = jax.lax.broadcasted_iota(jnp.int32, (GS, CAP), 1)
    hit0 = (a1 == e) & (iota_c == pos0.astype(jnp.int32)) & (pos0 < CAP)
    hit1 = (a2 == e) & (iota_c == pos1.astype(jnp.int32)) & (pos1 < CAP)
    disp = hit0.astype(jnp.float32) + hit1.astype(jnp.float32)
    comb_ref[0, 0] = jnp.where(hit0, m1, 0.0) + jnp.where(hit1, m2, 0.0)
    xe_ref[0] = jax.lax.dot_general(disp, xg, (((0,), (0,)), ((), ())),
                                    preferred_element_type=jnp.float32,
                                    precision=_HI)


def _ffn_body(xe_ref, w0_ref, w1_ref, wo_ref, oe_ref):
    mt = pl.program_id(1)
    a = xe_ref[0]
    h0 = jax.lax.dot_general(a, w0_ref[0], (((1,), (0,)), ((), ())),
                             preferred_element_type=jnp.float32)
    h1 = jax.lax.dot_general(a, w1_ref[0], (((1,), (0,)), ((), ())),
                             preferred_element_type=jnp.float32)
    h = (h0 * jax.lax.logistic(h0)) * h1
    out = jax.lax.dot_general(h, wo_ref[0], (((1,), (0,)), ((), ())),
                              preferred_element_type=jnp.float32)

    @pl.when(mt == 0)
    def _():
        oe_ref[0] = out

    @pl.when(mt > 0)
    def _():
        oe_ref[0] += out


def _combine_body(comb_ref, oe_ref, y_ref):
    e = pl.program_id(1)
    y = jax.lax.dot_general(comb_ref[0, 0], oe_ref[0],
                            (((1,), (0,)), ((), ())),
                            preferred_element_type=jnp.float32,
                            precision=_HI)

    @pl.when(e == 0)
    def _():
        y_ref[0] = y

    @pl.when(e > 0)
    def _():
        y_ref[0] += y


BM = 256


@jax.jit
def kernel(x, w_router, wi_0, wi_1, wo):
    xg = x.reshape(G, GS, D)

    xe, comb = pl.pallas_call(
        _router_body,
        grid=(G, E),
        in_specs=[
            pl.BlockSpec((1, GS, D), lambda g, e: (g, 0, 0)),
            pl.BlockSpec((D, E), lambda g, e: (0, 0)),
        ],
        out_specs=[
            pl.BlockSpec((1, CAP, D), lambda g, e: (e, g, 0)),
            pl.BlockSpec((1, 1, GS, CAP), lambda g, e: (g, e, 0, 0)),
        ],
        out_shape=[
            jax.ShapeDtypeStruct((E, G * CAP, D), jnp.float32),
            jax.ShapeDtypeStruct((G, E, GS, CAP), jnp.float32),
        ],
    )(xg, w_router)

    oe = pl.pallas_call(
        _ffn_body,
        grid=(E, M // BM),
        in_specs=[
            pl.BlockSpec((1, G * CAP, D), lambda e, mt: (e, 0, 0)),
            pl.BlockSpec((1, D, BM), lambda e, mt: (e, 0, mt)),
            pl.BlockSpec((1, D, BM), lambda e, mt: (e, 0, mt)),
            pl.BlockSpec((1, BM, D), lambda e, mt: (e, mt, 0)),
        ],
        out_specs=pl.BlockSpec((1, G * CAP, D), lambda e, mt: (e, 0, 0)),
        out_shape=jax.ShapeDtypeStruct((E, G * CAP, D), jnp.float32),
    )(xe, wi_0, wi_1, wo)

    y = pl.pallas_call(
        _combine_body,
        grid=(G, E),
        in_specs=[
            pl.BlockSpec((1, 1, GS, CAP), lambda g, e: (g, e, 0, 0)),
            pl.BlockSpec((1, CAP, D), lambda g, e: (e, g, 0)),
        ],
        out_specs=pl.BlockSpec((1, GS, D), lambda g, e: (g, 0, 0)),
        out_shape=jax.ShapeDtypeStruct((G, GS, D), jnp.float32),
    )(comb, oe)

    return y.reshape(B, S, D)


# speedup vs baseline: 4.0047x; 1.3824x over previous
import jax
import jax.numpy as jnp
from jax.experimental import pallas as pl

B, S, D = 1, 2048, 2048
E, K = 8, 2
M = 4096
G = 4
GS = (B * S) // G
CAP = int(GS * K / E * 1.25)


def _route(xg, wr):
    logits = jax.lax.dot_general(xg, wr, (((1,), (0,)), ((), ())),
                                 preferred_element_type=jnp.float32)
    lmax = jnp.max(logits, axis=1, keepdims=True)
    ex = jnp.exp(logits - lmax)
    probs = ex / jnp.sum(ex, axis=1, keepdims=True)

    iota_e = jax.lax.broadcasted_iota(jnp.int32, (GS, E), 1)
    m1 = jnp.max(probs, axis=1, keepdims=True)
    a1 = jnp.min(jnp.where(probs == m1, iota_e, E), axis=1, keepdims=True)
    probs2 = jnp.where(iota_e == a1, -jnp.inf, probs)
    m2 = jnp.max(probs2, axis=1, keepdims=True)
    a2 = jnp.min(jnp.where(probs2 == m2, iota_e, E), axis=1, keepdims=True)

    oh0 = (iota_e == a1).astype(jnp.float32)
    oh1 = (iota_e == a2).astype(jnp.float32)
    ri = jax.lax.broadcasted_iota(jnp.int32, (GS, GS), 0)
    ci = jax.lax.broadcasted_iota(jnp.int32, (GS, GS), 1)
    tri = (ci <= ri).astype(jnp.float32)
    c0 = jax.lax.dot_general(tri, oh0, (((1,), (0,)), ((), ())),
                             preferred_element_type=jnp.float32)
    c1 = jax.lax.dot_general(tri, oh1, (((1,), (0,)), ((), ())),
                             preferred_element_type=jnp.float32)
    total0 = c0[GS - 1:GS, :]
    pos0 = jnp.sum(c0 * oh0, axis=1, keepdims=True) - 1.0
    pos1 = jnp.sum((c1 + total0) * oh1, axis=1, keepdims=True) - 1.0
    return a1, a2, m1, m2, pos0, pos1


def _router_body(xg_ref, wr_ref, xe_ref, comb_ref):
    e = pl.program_id(1)
    xg = xg_ref[0]
    a1, a2, m1, m2, pos0, pos1 = _route(xg, wr_ref[...])

    iota_c = jax.lax.broadcasted_iota(jnp.int32, (GS, CAP), 1)
    hit0 = (a1 == e) & (iota_c == pos0.astype(jnp.int32)) & (pos0 < CAP)
    hit1 = (a2 == e) & (iota_c == pos1.astype(jnp.int32)) & (pos1 < CAP)
    disp = hit0.astype(jnp.float32) + hit1.astype(jnp.float32)
    comb_ref[0, 0] = jnp.where(hit0, m1, 0.0) + jnp.where(hit1, m2, 0.0)
    xe_ref[0] = jax.lax.dot_general(disp, xg, (((0,), (0,)), ((), ())),
                                    preferred_element_type=jnp.float32)


def _ffn_body(xe_ref, w0_ref, w1_ref, wo_ref, oe_ref):
    mt = pl.program_id(1)
    a = xe_ref[0]
    h0 = jax.lax.dot_general(a, w0_ref[0], (((1,), (0,)), ((), ())),
                             preferred_element_type=jnp.float32)
    h1 = jax.lax.dot_general(a, w1_ref[0], (((1,), (0,)), ((), ())),
                             preferred_element_type=jnp.float32)
    h = (h0 * jax.lax.logistic(h0)) * h1
    out = jax.lax.dot_general(h, wo_ref[0], (((1,), (0,)), ((), ())),
                              preferred_element_type=jnp.float32)

    @pl.when(mt == 0)
    def _():
        oe_ref[0] = out

    @pl.when(mt > 0)
    def _():
        oe_ref[0] += out


def _combine_body(comb_ref, oe_ref, y_ref):
    e = pl.program_id(1)
    y = jax.lax.dot_general(comb_ref[0, 0], oe_ref[0],
                            (((1,), (0,)), ((), ())),
                            preferred_element_type=jnp.float32)

    @pl.when(e == 0)
    def _():
        y_ref[0] = y

    @pl.when(e > 0)
    def _():
        y_ref[0] += y


BM = 512


@jax.jit
def kernel(x, w_router, wi_0, wi_1, wo):
    xg = x.reshape(G, GS, D)

    xe, comb = pl.pallas_call(
        _router_body,
        grid=(G, E),
        in_specs=[
            pl.BlockSpec((1, GS, D), lambda g, e: (g, 0, 0)),
            pl.BlockSpec((D, E), lambda g, e: (0, 0)),
        ],
        out_specs=[
            pl.BlockSpec((1, CAP, D), lambda g, e: (e, g, 0)),
            pl.BlockSpec((1, 1, GS, CAP), lambda g, e: (g, e, 0, 0)),
        ],
        out_shape=[
            jax.ShapeDtypeStruct((E, G * CAP, D), jnp.float32),
            jax.ShapeDtypeStruct((G, E, GS, CAP), jnp.float32),
        ],
    )(xg, w_router)

    oe = pl.pallas_call(
        _ffn_body,
        grid=(E, M // BM),
        in_specs=[
            pl.BlockSpec((1, G * CAP, D), lambda e, mt: (e, 0, 0)),
            pl.BlockSpec((1, D, BM), lambda e, mt: (e, 0, mt)),
            pl.BlockSpec((1, D, BM), lambda e, mt: (e, 0, mt)),
            pl.BlockSpec((1, BM, D), lambda e, mt: (e, mt, 0)),
        ],
        out_specs=pl.BlockSpec((1, G * CAP, D), lambda e, mt: (e, 0, 0)),
        out_shape=jax.ShapeDtypeStruct((E, G * CAP, D), jnp.float32),
    )(xe, wi_0, wi_1, wo)

    y = pl.pallas_call(
        _combine_body,
        grid=(G, E),
        in_specs=[
            pl.BlockSpec((1, 1, GS, CAP), lambda g, e: (g, e, 0, 0)),
            pl.BlockSpec((1, CAP, D), lambda g, e: (e, g, 0)),
        ],
        out_specs=pl.BlockSpec((1, GS, D), lambda g, e: (g, 0, 0)),
        out_shape=jax.ShapeDtypeStruct((G, GS, D), jnp.float32),
    )(comb, oe)

    return y.reshape(B, S, D)
